# speedup vs baseline: 2.1974x; 2.1974x over previous
import jax
import jax.numpy as jnp
from jax.experimental import pallas as pl

N_TOK = 32768
EMB = 64
USER = 16
EXPERTS = 64

BLK = 2048


def _gate_kernel(h_ref, u_ref, wgt_ref, bg_ref, wbt_ref, bb_ref, wlt_ref,
                 bl_ref, out_ref):
    h = h_ref[...]
    u = u_ref[...]
    gamma = jnp.dot(u, wgt_ref[...], preferred_element_type=jnp.float32)
    beta = jnp.dot(u, wbt_ref[...], preferred_element_type=jnp.float32)
    h_t = h * (1.0 + gamma + bg_ref[...]) + (beta + bb_ref[...])
    logits = jnp.dot(h_t, wlt_ref[...],
                     preferred_element_type=jnp.float32) + bl_ref[...]

    cols = jax.lax.broadcasted_iota(jnp.int32, logits.shape, 1)
    m1 = jnp.max(logits, axis=1, keepdims=True)
    i1 = jnp.min(jnp.where(logits == m1, cols, EXPERTS), axis=1,
                 keepdims=True)
    sel1 = cols == i1
    rest = jnp.where(sel1, -jnp.inf, logits)
    m2 = jnp.max(rest, axis=1, keepdims=True)
    i2 = jnp.min(jnp.where(rest == m2, cols, EXPERTS), axis=1, keepdims=True)
    mask = sel1 | (cols == i2)
    scale = 1.0 / (1.0 + jnp.exp(m2 - m1))
    out_ref[...] = jnp.where(mask, jnp.exp(logits - m1) * scale, 0.0)


@jax.jit
def _run(h, u, wgt, bg2, wbt, bb2, wlt, bl2):
    grid = (N_TOK // BLK,)
    tok_spec = lambda width: pl.BlockSpec((BLK, width), lambda i: (i, 0))
    full = lambda a: pl.BlockSpec(a.shape, lambda i: (0,) * a.ndim)
    return pl.pallas_call(
        _gate_kernel,
        grid=grid,
        in_specs=[
            tok_spec(EMB),
            tok_spec(USER),
            full(wgt), full(bg2), full(wbt), full(bb2), full(wlt), full(bl2),
        ],
        out_specs=tok_spec(EXPERTS),
        out_shape=jax.ShapeDtypeStruct((N_TOK, EXPERTS), jnp.float32),
    )(h, u, wgt, bg2, wbt, bb2, wlt, bl2)


def kernel(h, u, Wg, bg, Wb, bb, Wl, bl):
    return _run(h, u, Wg.T, bg.reshape(1, EMB), Wb.T, bb.reshape(1, EMB),
                Wl.T, bl.reshape(1, EXPERTS))
